# baseline (speedup 1.0000x reference)
import functools

import jax
import jax.numpy as jnp
from jax import lax
from jax.experimental import pallas as pl
from jax.experimental.pallas import tpu as pltpu
from jax.experimental.pallas import tpu_sc as plsc

B = 16384
F = 32
PACK = 128 // F
CHUNK = 128


def _extract_chunk(chunk_v, out_v, off_v, cbase, fuse_mul):
  lane = jnp.arange(16, dtype=jnp.int32)

  def group(g, carry):
    rows_l = g * 16 + lane
    offs = off_v[pl.ds(cbase + g * 16, 16)]
    for j in range(F):
      jv = jnp.full(16, j, dtype=jnp.int32)
      vals = plsc.load_gather(chunk_v, [rows_l, offs + j])
      if fuse_mul:
        prev = plsc.load_gather(out_v, [rows_l, jv])
        vals = vals * prev
      plsc.store_scatter(out_v, [rows_l, jv], vals)
    return carry

  lax.fori_loop(0, CHUNK // 16, group, 0)


@functools.lru_cache(maxsize=None)
def _make_sc_gather(nc: int, ns: int, b_per_w: int):
  mesh = plsc.VectorSubcoreMesh(core_axis_name="c", subcore_axis_name="s")

  @functools.partial(
      pl.kernel,
      mesh=mesh,
      out_type=(
          jax.ShapeDtypeStruct((B, F), jnp.float32),
          jax.ShapeDtypeStruct((B, F), jnp.float32),
          jax.ShapeDtypeStruct((B, F), jnp.float32),
      ),
      scratch_types=[
          pltpu.VMEM((b_per_w,), jnp.int32),
          pltpu.VMEM((b_per_w,), jnp.int32),
          pltpu.VMEM((b_per_w,), jnp.int32),
          pltpu.VMEM((b_per_w,), jnp.int32),
          pltpu.VMEM((b_per_w,), jnp.int32),
          pltpu.VMEM((b_per_w,), jnp.int32),
          pltpu.VMEM((CHUNK, 128), jnp.float32),
          pltpu.VMEM((CHUNK, F), jnp.float32),
          pltpu.VMEM((CHUNK, F), jnp.float32),
          pltpu.SemaphoreType.DMA,
      ],
      compiler_params=pltpu.CompilerParams(needs_layout_passes=False),
  )
  def sc_gather(uidx_hbm, iidx_hbm, gu_hbm, gi_hbm, mu_hbm, mi_hbm,
                gmf_out, mlpu_out, mlpi_out,
                uidx_v, iidx_v, urow_v, irow_v, uoff_v, ioff_v,
                chunk_v, ogmf_v, omlp_v, sem):
    wid = lax.axis_index("s") * nc + lax.axis_index("c")
    base = wid * b_per_w
    pltpu.sync_copy(uidx_hbm.at[pl.ds(base, b_per_w)], uidx_v)
    pltpu.sync_copy(iidx_hbm.at[pl.ds(base, b_per_w)], iidx_v)

    def split(i, carry):
      s = pl.ds(i * 16, 16)
      u = uidx_v[s]
      urow_v[s] = u >> 2
      uoff_v[s] = (u & 3) << 5
      it = iidx_v[s]
      irow_v[s] = it >> 2
      ioff_v[s] = (it & 3) << 5
      return carry

    lax.fori_loop(0, b_per_w // 16, split, 0)

    n_chunks = b_per_w // CHUNK
    for c in range(n_chunks):
      cb = c * CHUNK
      out_slice = pl.ds(base + cb, CHUNK)
      for table, row_v, off_v, out_v, fuse, out_hbm in (
          (gu_hbm, urow_v, uoff_v, ogmf_v, False, None),
          (gi_hbm, irow_v, ioff_v, ogmf_v, True, gmf_out),
          (mu_hbm, urow_v, uoff_v, omlp_v, False, mlpu_out),
          (mi_hbm, irow_v, ioff_v, omlp_v, False, mlpi_out),
      ):
        idx_slice = row_v.at[pl.ds(cb, CHUNK)]
        pltpu.async_copy(table.at[idx_slice], chunk_v, sem).wait()
        _extract_chunk(chunk_v, out_v, off_v, cb, fuse)
        if out_hbm is not None:
          pltpu.sync_copy(out_v, out_hbm.at[out_slice])

  return sc_gather


def _tc_mlp_body(mu_ref, mi_ref, gmf_ref, w1a_ref, w1b_ref, b1_ref,
                 w2_ref, b2_ref, wog_ref, wom_ref, bo_ref, out_ref):
  h = jnp.dot(mu_ref[...], w1a_ref[...], preferred_element_type=jnp.float32)
  h = h + jnp.dot(mi_ref[...], w1b_ref[...], preferred_element_type=jnp.float32)
  h = jnp.maximum(h + b1_ref[...], 0.0)
  h2 = jnp.dot(h, w2_ref[...], preferred_element_type=jnp.float32)
  h2 = jnp.maximum(h2 + b2_ref[...], 0.0)
  out = jnp.sum(gmf_ref[...] * wog_ref[...], axis=1)
  out = out + jnp.sum(h2 * wom_ref[...], axis=1)
  out_ref[...] = out + bo_ref[0]


def _tc_mlp(mlp_u, mlp_i, gmf, W1a, W1b, b1, W2, b2, wo_g, wo_m, bo):
  blk = 2048
  grid = (B // blk,)
  row_spec = pl.BlockSpec((blk, F), lambda i: (i, 0))
  full = lambda shape: pl.BlockSpec(shape, lambda i: tuple(0 for _ in shape))
  return pl.pallas_call(
      _tc_mlp_body,
      grid=grid,
      in_specs=[
          row_spec, row_spec, row_spec,
          full(W1a.shape), full(W1b.shape), full(b1.shape),
          full(W2.shape), full(b2.shape),
          full(wo_g.shape), full(wo_m.shape), full(bo.shape),
      ],
      out_specs=pl.BlockSpec((blk,), lambda i: (i,)),
      out_shape=jax.ShapeDtypeStruct((B,), jnp.float32),
  )(mlp_u, mlp_i, gmf, W1a, W1b, b1, W2, b2, wo_g, wo_m, bo)


@jax.jit
def _neumf(user_idx, item_idx, gmf_user_emb, gmf_item_emb,
           mlp_user_emb, mlp_item_emb, W1, b1, W2, b2, Wo, bo):
  info = plsc.get_sparse_core_info()
  nw = info.num_cores * info.num_subcores
  sc = _make_sc_gather(info.num_cores, info.num_subcores, B // nw)
  gmf, mlp_u, mlp_i = sc(
      user_idx.astype(jnp.int32), item_idx.astype(jnp.int32),
      gmf_user_emb.reshape(-1, 128), gmf_item_emb.reshape(-1, 128),
      mlp_user_emb.reshape(-1, 128), mlp_item_emb.reshape(-1, 128))
  W1a, W1b = W1[:F], W1[F:]
  wo_g, wo_m = Wo[:F, 0], Wo[F:, 0]
  return _tc_mlp(mlp_u, mlp_i, gmf, W1a, W1b, b1, W2, b2, wo_g, wo_m, bo)


def kernel(user_idx, item_idx, gmf_user_emb, gmf_item_emb,
           mlp_user_emb, mlp_item_emb, W1, b1, W2, b2, Wo, bo):
  return _neumf(user_idx, item_idx, gmf_user_emb, gmf_item_emb,
                mlp_user_emb, mlp_item_emb, W1, b1, W2, b2, Wo, bo)
